# Initial kernel scaffold; baseline (speedup 1.0000x reference)
import jax
import jax.numpy as jnp
from jax.experimental import pallas as pl


def kernel(x):
    raise NotImplementedError("write your pallas kernel here")


# speedup vs baseline: 2.1598x; 2.1598x over previous
import functools

import jax
import jax.numpy as jnp
from jax import lax
from jax.experimental import pallas as pl
from jax.experimental.pallas import tpu as pltpu
from jax.experimental.pallas import tpu_sc as plsc

R = 128
N = 32768
NW = 32
ROWS_PW = R // NW
L = 16
CH = N // L
DIGIT_BITS = 11
NB = 1 << DIGIT_BITS
MIN32 = jnp.int32(-(1 << 31))


def _to_mono(v):
    return v ^ ((v >> 31) | MIN32)


def _from_mono(u):
    return u ^ jnp.where(u < 0, MIN32, jnp.int32(-1))


def _digit(u, shift):
    uu = plsc.bitcast(u, jnp.uint32)
    d = (uu >> shift) & jnp.uint32(NB - 1)
    return plsc.bitcast(d, jnp.int32)


def _radix_pass(src, dst, hist, shift, first, final):
    def zero_body(i, c):
        hist[pl.ds(i * L, L)] = jnp.zeros((L,), jnp.int32)
        return c

    lax.fori_loop(0, NB // L, zero_body, 0)

    def hist_body(i, c):
        v = src[pl.ds(i * L, L)]
        u = _to_mono(v) if first else v
        d = _digit(u, shift)
        cnt, lastm = plsc.scan_count(d)
        plsc.addupdate_scatter(hist, [d], cnt, mask=lastm)
        return c

    lax.fori_loop(0, CH, hist_body, 0)

    def scan_body(i, carry):
        v = hist[pl.ds(i * L, L)]
        inc = plsc.cumsum(v)
        hist[pl.ds(i * L, L)] = inc - v + carry
        return carry + jnp.sum(v)

    lax.fori_loop(0, NB // L, scan_body, jnp.int32(0))

    def perm_body(i, c):
        v = src[pl.ds(i * L, L)]
        u = _to_mono(v) if first else v
        d = _digit(u, shift)
        cnt, lastm = plsc.scan_count(d)
        base = plsc.load_gather(hist, [d])
        pos = base + cnt - 1
        w = _from_mono(u) if final else u
        plsc.store_scatter(dst, [pos], w)
        plsc.addupdate_scatter(hist, [d], cnt, mask=lastm)
        return c

    lax.fori_loop(0, CH, perm_body, 0)


_mesh = plsc.VectorSubcoreMesh(core_axis_name="c", subcore_axis_name="s")


@functools.partial(
    pl.kernel,
    mesh=_mesh,
    compiler_params=pltpu.CompilerParams(needs_layout_passes=False),
    out_type=jax.ShapeDtypeStruct((R, N), jnp.int32),
    scratch_types=[
        pltpu.VMEM((N,), jnp.int32),
        pltpu.VMEM((N,), jnp.int32),
        pltpu.VMEM((NB,), jnp.int32),
    ],
)
def _sort_rows(x_hbm, out_hbm, buf_a, buf_b, hist):
    wid = lax.axis_index("s") * 2 + lax.axis_index("c")
    for r in range(ROWS_PW):
        row = wid * ROWS_PW + r
        pltpu.sync_copy(x_hbm.at[row], buf_a)
        _radix_pass(buf_a, buf_b, hist, 0, True, False)
        _radix_pass(buf_b, buf_a, hist, DIGIT_BITS, False, False)
        _radix_pass(buf_a, buf_b, hist, 2 * DIGIT_BITS, False, True)
        pltpu.sync_copy(buf_b, out_hbm.at[row])


def kernel(x):
    xi = lax.bitcast_convert_type(x, jnp.int32)
    yi = _sort_rows(xi)
    return lax.bitcast_convert_type(yi, jnp.float32)


# speedup vs baseline: 2.2518x; 1.0426x over previous
import functools

import jax
import jax.numpy as jnp
from jax import lax
from jax.experimental import pallas as pl
from jax.experimental.pallas import tpu as pltpu
from jax.experimental.pallas import tpu_sc as plsc

R = 128
N = 32768
NW = 32
ROWS_PW = R // NW
L = 16
CH = N // L
DIGIT_BITS = 11
NB = 1 << DIGIT_BITS
MIN32 = jnp.int32(-(1 << 31))


def _to_mono(v):
    return v ^ ((v >> 31) | MIN32)


def _from_mono(u):
    return u ^ jnp.where(u < 0, MIN32, jnp.int32(-1))


def _digit(u, shift):
    uu = plsc.bitcast(u, jnp.uint32)
    d = (uu >> shift) & jnp.uint32(NB - 1)
    return plsc.bitcast(d, jnp.int32)


def _radix_pass(src, dst, hist, shift, first, final):
    def zero_body(i, c):
        hist[pl.ds(i * L, L)] = jnp.zeros((L,), jnp.int32)
        return c

    lax.fori_loop(0, NB // L, zero_body, 0, unroll=8)

    def hist_body(i, c):
        v = src[pl.ds(i * L, L)]
        u = _to_mono(v) if first else v
        d = _digit(u, shift)
        cnt, lastm = plsc.scan_count(d)
        plsc.addupdate_scatter(hist, [d], cnt, mask=lastm)
        return c

    lax.fori_loop(0, CH, hist_body, 0, unroll=8)

    def scan_body(i, carry):
        v = hist[pl.ds(i * L, L)]
        inc = plsc.cumsum(v)
        hist[pl.ds(i * L, L)] = inc - v + carry
        return carry + jnp.sum(v)

    lax.fori_loop(0, NB // L, scan_body, jnp.int32(-1), unroll=4)

    def perm_body(i, c):
        v = src[pl.ds(i * L, L)]
        u = _to_mono(v) if first else v
        d = _digit(u, shift)
        cnt, lastm = plsc.scan_count(d)
        base = plsc.load_gather(hist, [d])
        pos = base + cnt
        w = _from_mono(u) if final else u
        plsc.store_scatter(dst, [pos], w)
        plsc.addupdate_scatter(hist, [d], cnt, mask=lastm)
        return c

    lax.fori_loop(0, CH, perm_body, 0, unroll=8)


_mesh = plsc.VectorSubcoreMesh(core_axis_name="c", subcore_axis_name="s")


@functools.partial(
    pl.kernel,
    mesh=_mesh,
    compiler_params=pltpu.CompilerParams(needs_layout_passes=False),
    out_type=jax.ShapeDtypeStruct((R, N), jnp.int32),
    scratch_types=[
        pltpu.VMEM((N,), jnp.int32),
        pltpu.VMEM((N,), jnp.int32),
        pltpu.VMEM((NB,), jnp.int32),
    ],
)
def _sort_rows(x_hbm, out_hbm, buf_a, buf_b, hist):
    wid = lax.axis_index("s") * 2 + lax.axis_index("c")
    for r in range(ROWS_PW):
        row = wid * ROWS_PW + r
        pltpu.sync_copy(x_hbm.at[row], buf_a)
        _radix_pass(buf_a, buf_b, hist, 0, True, False)
        _radix_pass(buf_b, buf_a, hist, DIGIT_BITS, False, False)
        _radix_pass(buf_a, buf_b, hist, 2 * DIGIT_BITS, False, True)
        pltpu.sync_copy(buf_b, out_hbm.at[row])


def kernel(x):
    xi = lax.bitcast_convert_type(x, jnp.int32)
    yi = _sort_rows(xi)
    return lax.bitcast_convert_type(yi, jnp.float32)


# speedup vs baseline: 3.1667x; 1.4063x over previous
import functools

import jax
import jax.numpy as jnp
from jax import lax
from jax.experimental import pallas as pl
from jax.experimental.pallas import tpu as pltpu
from jax.experimental.pallas import tpu_sc as plsc

R = 128
N = 32768
NW = 32
ROWS_PW = R // NW
L = 16
CH = N // L
DB = 11
NB = 1 << DB
NB3 = 1 << (32 - 2 * DB)
MIN32 = jnp.int32(-(1 << 31))
UN = 8


def _to_mono(v):
    return v ^ ((v >> 31) | MIN32)


def _from_mono(u):
    return u ^ jnp.where(u < 0, MIN32, jnp.int32(-1))


def _digit(u, shift, mask_bits):
    uu = plsc.bitcast(u, jnp.uint32)
    uu = uu >> shift if shift else uu
    if mask_bits:
        uu = uu & jnp.uint32(mask_bits)
    return plsc.bitcast(uu, jnp.int32)


def _zero(h, nb):
    def body(i, c):
        h[pl.ds(i * L, L)] = jnp.zeros((L,), jnp.int32)
        return c

    lax.fori_loop(0, nb // L, body, 0, unroll=UN)


def _exscan(h, nb):
    def body(i, carry):
        v = h[pl.ds(i * L, L)]
        inc = plsc.cumsum(v)
        h[pl.ds(i * L, L)] = inc - v + carry
        return carry + jnp.sum(v)

    lax.fori_loop(0, nb // L, body, jnp.int32(-1), unroll=4)


def _hist_add(h, d):
    cnt, lastm = plsc.scan_count(d)
    plsc.addupdate_scatter(h, [d], cnt, mask=lastm)


def _place(h, d, dst, w):
    cnt, lastm = plsc.scan_count(d)
    base = plsc.load_gather(h, [d])
    plsc.store_scatter(dst, [base + cnt], w)
    plsc.addupdate_scatter(h, [d], cnt, mask=lastm)


_mesh = plsc.VectorSubcoreMesh(core_axis_name="c", subcore_axis_name="s")


@functools.partial(
    pl.kernel,
    mesh=_mesh,
    compiler_params=pltpu.CompilerParams(needs_layout_passes=False),
    out_type=jax.ShapeDtypeStruct((R, N), jnp.int32),
    scratch_types=[
        pltpu.VMEM((N,), jnp.int32),
        pltpu.VMEM((N,), jnp.int32),
        pltpu.VMEM((NB,), jnp.int32),
        pltpu.VMEM((NB,), jnp.int32),
        pltpu.VMEM((NB3,), jnp.int32),
    ],
)
def _sort_rows(x_hbm, out_hbm, buf_a, buf_b, h1, h2, h3):
    wid = lax.axis_index("s") * 2 + lax.axis_index("c")
    for r in range(ROWS_PW):
        row = wid * ROWS_PW + r
        pltpu.sync_copy(x_hbm.at[row], buf_a)

        _zero(h1, NB)

        def hist1(i, c):
            u = _to_mono(buf_a[pl.ds(i * L, L)])
            _hist_add(h1, _digit(u, 0, NB - 1))
            return c

        lax.fori_loop(0, CH, hist1, 0, unroll=UN)
        _exscan(h1, NB)
        _zero(h2, NB)

        def perm1(i, c):
            u = _to_mono(buf_a[pl.ds(i * L, L)])
            _place(h1, _digit(u, 0, NB - 1), buf_b, u)
            _hist_add(h2, _digit(u, DB, NB - 1))
            return c

        lax.fori_loop(0, CH, perm1, 0, unroll=UN)
        _exscan(h2, NB)
        _zero(h3, NB3)

        def perm2(i, c):
            u = buf_b[pl.ds(i * L, L)]
            _place(h2, _digit(u, DB, NB - 1), buf_a, u)
            _hist_add(h3, _digit(u, 2 * DB, 0))
            return c

        lax.fori_loop(0, CH, perm2, 0, unroll=UN)
        _exscan(h3, NB3)

        def perm3(i, c):
            u = buf_a[pl.ds(i * L, L)]
            d = _digit(u, 2 * DB, 0)
            cnt, lastm = plsc.scan_count(d)
            base = plsc.load_gather(h3, [d])
            plsc.store_scatter(buf_b, [base + cnt], _from_mono(u))
            plsc.addupdate_scatter(h3, [d], cnt, mask=lastm)
            return c

        lax.fori_loop(0, CH, perm3, 0, unroll=UN)
        pltpu.sync_copy(buf_b, out_hbm.at[row])


def kernel(x):
    xi = lax.bitcast_convert_type(x, jnp.int32)
    yi = _sort_rows(xi)
    return lax.bitcast_convert_type(yi, jnp.float32)


# speedup vs baseline: 5.2909x; 1.6708x over previous
import functools

import jax
import jax.numpy as jnp
from jax import lax
from jax.experimental import pallas as pl
from jax.experimental.pallas import tpu as pltpu
from jax.experimental.pallas import tpu_sc as plsc

R = 128
N = 32768
NW = 32
ROWS_PW = R // NW
L = 16
CH = N // L
DB = 11
NB = 1 << DB
NB3 = 1 << (32 - 2 * DB)
MIN32 = -(1 << 31)
UN = 8


def _to_mono(v):
    return v ^ ((v >> 31) | MIN32)


def _from_mono(u):
    return u ^ jnp.where(u < 0, MIN32, -1)


def _digit(u, shift, mask_bits):
    uu = plsc.bitcast(u, jnp.uint32)
    uu = uu >> shift if shift else uu
    if mask_bits:
        uu = uu & mask_bits
    return plsc.bitcast(uu, jnp.int32)


def _zero(h, nb):
    def body(i, c):
        h[pl.ds(i * L, L)] = jnp.zeros((L,), jnp.int32)
        return c

    lax.fori_loop(0, nb // L, body, 0, unroll=UN)


def _exscan(h, nb):
    def body(i, carry):
        v = h[pl.ds(i * L, L)]
        inc = plsc.cumsum(v)
        h[pl.ds(i * L, L)] = inc - v + carry
        return carry + jnp.sum(v)

    lax.fori_loop(0, nb // L, body, jnp.full((), -1, jnp.int32), unroll=4)


def _pipe(n, prefetch, commit):
    def body(i, st):
        nxt = prefetch(i + 1)
        commit(st)
        return nxt

    last = lax.fori_loop(0, n - 1, body, prefetch(0), unroll=UN)
    commit(last)


_mesh = plsc.VectorSubcoreMesh(core_axis_name="c", subcore_axis_name="s")


@functools.partial(
    pl.kernel,
    mesh=_mesh,
    compiler_params=pltpu.CompilerParams(needs_layout_passes=False),
    out_type=jax.ShapeDtypeStruct((R, N), jnp.int32),
    scratch_types=[
        pltpu.VMEM((N,), jnp.int32),
        pltpu.VMEM((N,), jnp.int32),
        pltpu.VMEM((NB,), jnp.int32),
        pltpu.VMEM((NB,), jnp.int32),
        pltpu.VMEM((NB3,), jnp.int32),
    ],
)
def _sort_rows(x_hbm, out_hbm, buf_a, buf_b, h1, h2, h3):
    wid = lax.axis_index("s") * 2 + lax.axis_index("c")
    for r in range(ROWS_PW):
        row = wid * ROWS_PW + r
        pltpu.sync_copy(x_hbm.at[row], buf_a)

        _zero(h1, NB)

        def pre_h1(i):
            u = _to_mono(buf_a[pl.ds(i * L, L)])
            d = _digit(u, 0, NB - 1)
            cnt, m = plsc.scan_count(d)
            return d, cnt, m

        def com_h1(st):
            d, cnt, m = st
            plsc.addupdate_scatter(h1, [d], cnt, mask=m)

        _pipe(CH, pre_h1, com_h1)
        _exscan(h1, NB)
        _zero(h2, NB)

        def pre_p1(i):
            u = _to_mono(buf_a[pl.ds(i * L, L)])
            d1 = _digit(u, 0, NB - 1)
            d2 = _digit(u, DB, NB - 1)
            c1, m1 = plsc.scan_count(d1)
            c2, m2 = plsc.scan_count(d2)
            return u, d1, d2, c1, m1, c2, m2

        def com_p1(st):
            u, d1, d2, c1, m1, c2, m2 = st
            base = plsc.load_gather(h1, [d1])
            plsc.store_scatter(buf_b, [base + c1], u)
            plsc.addupdate_scatter(h1, [d1], c1, mask=m1)
            plsc.addupdate_scatter(h2, [d2], c2, mask=m2)

        _pipe(CH, pre_p1, com_p1)
        _exscan(h2, NB)
        _zero(h3, NB3)

        def pre_p2(i):
            u = buf_b[pl.ds(i * L, L)]
            d2 = _digit(u, DB, NB - 1)
            d3 = _digit(u, 2 * DB, 0)
            c2, m2 = plsc.scan_count(d2)
            c3, m3 = plsc.scan_count(d3)
            return u, d2, d3, c2, m2, c3, m3

        def com_p2(st):
            u, d2, d3, c2, m2, c3, m3 = st
            base = plsc.load_gather(h2, [d2])
            plsc.store_scatter(buf_a, [base + c2], u)
            plsc.addupdate_scatter(h2, [d2], c2, mask=m2)
            plsc.addupdate_scatter(h3, [d3], c3, mask=m3)

        _pipe(CH, pre_p2, com_p2)
        _exscan(h3, NB3)

        def pre_p3(i):
            u = buf_a[pl.ds(i * L, L)]
            d3 = _digit(u, 2 * DB, 0)
            c3, m3 = plsc.scan_count(d3)
            return u, d3, c3, m3

        def com_p3(st):
            u, d3, c3, m3 = st
            base = plsc.load_gather(h3, [d3])
            plsc.store_scatter(buf_b, [base + c3], _from_mono(u))
            plsc.addupdate_scatter(h3, [d3], c3, mask=m3)

        _pipe(CH, pre_p3, com_p3)
        pltpu.sync_copy(buf_b, out_hbm.at[row])


def kernel(x):
    xi = lax.bitcast_convert_type(x, jnp.int32)
    yi = _sort_rows(xi)
    return lax.bitcast_convert_type(yi, jnp.float32)


# speedup vs baseline: 5.6957x; 1.0765x over previous
import functools

import jax
import jax.numpy as jnp
from jax import lax
from jax.experimental import pallas as pl
from jax.experimental.pallas import tpu as pltpu
from jax.experimental.pallas import tpu_sc as plsc

R = 128
N = 32768
NW = 32
ROWS_PW = R // NW
L = 16
CH = N // L
DB = 11
NB = 1 << DB
NB3 = 1 << (32 - 2 * DB)
MIN32 = -(1 << 31)
UN = 8


def _to_mono(v):
    return v ^ ((v >> 31) | MIN32)


def _from_mono(u):
    return u ^ jnp.where(u < 0, MIN32, -1)


def _digit(u, shift, mask_bits):
    uu = plsc.bitcast(u, jnp.uint32)
    uu = uu >> shift if shift else uu
    if mask_bits:
        uu = uu & mask_bits
    return plsc.bitcast(uu, jnp.int32)


def _zero(h, nb):
    def body(i, c):
        h[pl.ds(i * L, L)] = jnp.zeros((L,), jnp.int32)
        return c

    lax.fori_loop(0, nb // L, body, 0, unroll=UN)


def _exscan(h, nb):
    def body(i, carry):
        v = h[pl.ds(i * L, L)]
        inc = plsc.cumsum(v)
        h[pl.ds(i * L, L)] = inc - v + carry
        return carry + jnp.sum(v)

    lax.fori_loop(0, nb // L, body, jnp.full((), -1, jnp.int32), unroll=4)


def _ones():
    return jnp.full((L,), 1, jnp.int32)


def _pipe(n, prefetch, commit):
    def body(i, st):
        nxt = prefetch(i + 1)
        commit(st)
        return nxt

    last = lax.fori_loop(0, n - 1, body, prefetch(0), unroll=UN)
    commit(last)


_mesh = plsc.VectorSubcoreMesh(core_axis_name="c", subcore_axis_name="s")


@functools.partial(
    pl.kernel,
    mesh=_mesh,
    compiler_params=pltpu.CompilerParams(needs_layout_passes=False),
    out_type=jax.ShapeDtypeStruct((R, N), jnp.int32),
    scratch_types=[
        pltpu.VMEM((N,), jnp.int32),
        pltpu.VMEM((N,), jnp.int32),
        pltpu.VMEM((NB,), jnp.int32),
        pltpu.VMEM((NB,), jnp.int32),
        pltpu.VMEM((NB3,), jnp.int32),
    ],
)
def _sort_rows(x_hbm, out_hbm, buf_a, buf_b, h1, h2, h3):
    wid = lax.axis_index("s") * 2 + lax.axis_index("c")
    for r in range(ROWS_PW):
        row = wid * ROWS_PW + r
        pltpu.sync_copy(x_hbm.at[row], buf_a)

        _zero(h1, NB)

        def pre_h1(i):
            u = _to_mono(buf_a[pl.ds(i * L, L)])
            buf_a[pl.ds(i * L, L)] = u
            return (_digit(u, 0, NB - 1),)

        def com_h1(st):
            plsc.addupdate_scatter(h1, [st[0]], _ones())

        _pipe(CH, pre_h1, com_h1)
        _exscan(h1, NB)
        _zero(h2, NB)

        def pre_p1(i):
            u = buf_a[pl.ds(i * L, L)]
            d1 = _digit(u, 0, NB - 1)
            d2 = _digit(u, DB, NB - 1)
            c1, m1 = plsc.scan_count(d1)
            return u, d1, d2, c1, m1

        def com_p1(st):
            u, d1, d2, c1, m1 = st
            base = plsc.load_gather(h1, [d1])
            plsc.store_scatter(buf_b, [base + c1], u)
            plsc.addupdate_scatter(h1, [d1], c1, mask=m1)
            plsc.addupdate_scatter(h2, [d2], _ones())

        _pipe(CH, pre_p1, com_p1)
        _exscan(h2, NB)
        _zero(h3, NB3)

        def pre_p2(i):
            u = buf_b[pl.ds(i * L, L)]
            d2 = _digit(u, DB, NB - 1)
            d3 = _digit(u, 2 * DB, 0)
            c2, m2 = plsc.scan_count(d2)
            return u, d2, d3, c2, m2

        def com_p2(st):
            u, d2, d3, c2, m2 = st
            base = plsc.load_gather(h2, [d2])
            plsc.store_scatter(buf_a, [base + c2], u)
            plsc.addupdate_scatter(h2, [d2], c2, mask=m2)
            plsc.addupdate_scatter(h3, [d3], _ones())

        _pipe(CH, pre_p2, com_p2)
        _exscan(h3, NB3)

        def pre_p3(i):
            u = buf_a[pl.ds(i * L, L)]
            d3 = _digit(u, 2 * DB, 0)
            c3, m3 = plsc.scan_count(d3)
            return u, d3, c3, m3

        def com_p3(st):
            u, d3, c3, m3 = st
            base = plsc.load_gather(h3, [d3])
            plsc.store_scatter(buf_b, [base + c3], _from_mono(u))
            plsc.addupdate_scatter(h3, [d3], c3, mask=m3)

        _pipe(CH, pre_p3, com_p3)
        pltpu.sync_copy(buf_b, out_hbm.at[row])


def kernel(x):
    xi = lax.bitcast_convert_type(x, jnp.int32)
    yi = _sort_rows(xi)
    return lax.bitcast_convert_type(yi, jnp.float32)


# speedup vs baseline: 5.8325x; 1.0240x over previous
import functools

import jax
import jax.numpy as jnp
from jax import lax
from jax.experimental import pallas as pl
from jax.experimental.pallas import tpu as pltpu
from jax.experimental.pallas import tpu_sc as plsc

R = 128
N = 32768
NW = 32
ROWS_PW = R // NW
L = 16
CH = N // L
DB = 11
NB = 1 << DB
NB3 = 1 << (32 - 2 * DB)
MIN32 = -(1 << 31)
UN = 8


def _to_mono(v):
    return v ^ ((v >> 31) | MIN32)


def _from_mono(u):
    return u ^ jnp.where(u < 0, MIN32, -1)


def _digit(u, shift, mask_bits):
    uu = plsc.bitcast(u, jnp.uint32)
    uu = uu >> shift if shift else uu
    if mask_bits:
        uu = uu & mask_bits
    return plsc.bitcast(uu, jnp.int32)


def _zero(h, nb):
    def body(i, c):
        h[pl.ds(i * L, L)] = jnp.zeros((L,), jnp.int32)
        return c

    lax.fori_loop(0, nb // L, body, 0, unroll=UN)


def _exscan(h, nb):
    def body(i, carry):
        v = h[pl.ds(i * L, L)]
        inc = plsc.cumsum(v)
        h[pl.ds(i * L, L)] = inc - v + carry
        return carry + jnp.sum(v)

    lax.fori_loop(0, nb // L, body, jnp.full((), -1, jnp.int32), unroll=4)


def _ones():
    return jnp.full((L,), 1, jnp.int32)


def _pipe(n, prefetch, commit):
    def body(i, st):
        nxt = prefetch(i + 1)
        commit(st)
        return nxt

    last = lax.fori_loop(0, n - 1, body, prefetch(0), unroll=UN)
    commit(last)


_mesh = plsc.VectorSubcoreMesh(core_axis_name="c", subcore_axis_name="s")


@functools.partial(
    pl.kernel,
    mesh=_mesh,
    compiler_params=pltpu.CompilerParams(needs_layout_passes=False),
    out_type=jax.ShapeDtypeStruct((R, N), jnp.int32),
    scratch_types=[
        pltpu.VMEM((N,), jnp.int32),
        pltpu.VMEM((N,), jnp.int32),
        pltpu.VMEM((N,), jnp.int32),
        pltpu.VMEM((NB,), jnp.int32),
        pltpu.VMEM((NB,), jnp.int32),
        pltpu.VMEM((NB3,), jnp.int32),
        pltpu.SemaphoreType.DMA,
        pltpu.SemaphoreType.DMA,
    ],
)
def _sort_rows(x_hbm, out_hbm, b0, b1, b2, h1, h2, h3, in_sem, out_sem):
    wid = lax.axis_index("s") * 2 + lax.axis_index("c")
    bufs = (b0, b1, b2)
    rot = [(0, 1, 2), (2, 0, 1), (1, 2, 0), (0, 1, 2)]
    in_h = None
    out_hs = []
    for r in range(ROWS_PW):
        row = wid * ROWS_PW + r
        buf_a = bufs[rot[r][0]]
        buf_b = bufs[rot[r][1]]
        buf_s = bufs[rot[r][2]]
        if r == 0:
            pltpu.sync_copy(x_hbm.at[row], buf_a)
        else:
            in_h.wait()
        if r + 1 < ROWS_PW:
            if out_hs:
                out_hs.pop(0).wait()
            in_h = pltpu.async_copy(x_hbm.at[row + 1], buf_s, in_sem)

        _zero(h1, NB)

        def pre_h1(i):
            u = _to_mono(buf_a[pl.ds(i * L, L)])
            buf_a[pl.ds(i * L, L)] = u
            return (_digit(u, 0, NB - 1),)

        def com_h1(st):
            plsc.addupdate_scatter(h1, [st[0]], _ones())

        _pipe(CH, pre_h1, com_h1)
        _exscan(h1, NB)
        _zero(h2, NB)

        def pre_p1(i):
            u = buf_a[pl.ds(i * L, L)]
            d1 = _digit(u, 0, NB - 1)
            d2 = _digit(u, DB, NB - 1)
            c1, m1 = plsc.scan_count(d1)
            return u, d1, d2, c1, m1

        def com_p1(st):
            u, d1, d2, c1, m1 = st
            base = plsc.load_gather(h1, [d1])
            plsc.store_scatter(buf_b, [base + c1], u)
            plsc.addupdate_scatter(h1, [d1], c1, mask=m1)
            plsc.addupdate_scatter(h2, [d2], _ones())

        _pipe(CH, pre_p1, com_p1)
        _exscan(h2, NB)
        _zero(h3, NB3)

        def pre_p2(i):
            u = buf_b[pl.ds(i * L, L)]
            d2 = _digit(u, DB, NB - 1)
            d3 = _digit(u, 2 * DB, 0)
            c2, m2 = plsc.scan_count(d2)
            return u, d2, d3, c2, m2

        def com_p2(st):
            u, d2, d3, c2, m2 = st
            base = plsc.load_gather(h2, [d2])
            plsc.store_scatter(buf_a, [base + c2], u)
            plsc.addupdate_scatter(h2, [d2], c2, mask=m2)
            plsc.addupdate_scatter(h3, [d3], _ones())

        _pipe(CH, pre_p2, com_p2)
        _exscan(h3, NB3)

        def pre_p3(i):
            u = buf_a[pl.ds(i * L, L)]
            d3 = _digit(u, 2 * DB, 0)
            c3, m3 = plsc.scan_count(d3)
            return u, d3, c3, m3

        def com_p3(st):
            u, d3, c3, m3 = st
            base = plsc.load_gather(h3, [d3])
            plsc.store_scatter(buf_b, [base + c3], _from_mono(u))
            plsc.addupdate_scatter(h3, [d3], c3, mask=m3)

        _pipe(CH, pre_p3, com_p3)
        out_hs.append(pltpu.async_copy(buf_b, out_hbm.at[row], out_sem))
    for h in out_hs:
        h.wait()


def kernel(x):
    xi = lax.bitcast_convert_type(x, jnp.int32)
    yi = _sort_rows(xi)
    return lax.bitcast_convert_type(yi, jnp.float32)
